# baseline (speedup 1.0000x reference)
import functools

import jax
import jax.numpy as jnp
from jax import lax
from jax.experimental import pallas as pl
from jax.experimental.pallas import tpu as pltpu
from jax.experimental.pallas import tpu_sc as plsc

NC = 2
NS = 16
NW = NC * NS
BW = 512


@functools.partial(jax.jit, static_argnums=(0, 1, 2, 3))
def _run(B, F, E, NF, xf, vf, wt):
    ntile = B // 128
    nbt = BW // 128
    ET = E // 8
    mesh = plsc.VectorSubcoreMesh(
        core_axis_name="c", subcore_axis_name="s",
        num_cores=NC, num_subcores=NS)

    @functools.partial(
        pl.kernel,
        out_type=jax.ShapeDtypeStruct((F * ET * ntile, 8, 128),
                                      jnp.float32),
        mesh=mesh,
        scratch_types=[
            pltpu.VMEM((F * BW,), jnp.int32),
            pltpu.VMEM((F * BW,), jnp.float32),
            pltpu.VMEM((E, BW), jnp.float32),
            pltpu.SemaphoreType.DMA,
            pltpu.SemaphoreType.DMA,
            pltpu.SemaphoreType.DMA,
        ],
        compiler_params=pltpu.CompilerParams(use_tc_tiling_on_sc=False),
    )
    def emb_kernel(xf_hbm, vf_hbm, wt_hbm, out_hbm,
                   idx_all, v_all, stg, sem_l, sem_g, sem_o):
        wid = lax.axis_index("s") * NC + lax.axis_index("c")
        b0 = wid * BW

        for f in range(F):
            pltpu.async_copy(xf_hbm.at[pl.ds(f * B + b0, BW)],
                             idx_all.at[pl.ds(f * BW, BW)], sem_l)
            pltpu.async_copy(vf_hbm.at[pl.ds(f * B + b0, BW)],
                             v_all.at[pl.ds(f * BW, BW)], sem_l)
        pltpu.make_async_copy(xf_hbm.at[pl.ds(0, F * BW)], idx_all,
                              sem_l).wait()
        pltpu.make_async_copy(vf_hbm.at[pl.ds(0, F * BW)], v_all,
                              sem_l).wait()

        def f_body(f, carry):
            idx_f = idx_all.at[pl.ds(f * BW, BW)]
            descs = []
            for e in range(E):
                descs.append(pltpu.async_copy(
                    wt_hbm.at[e].at[idx_f], stg.at[e], sem_g))
            for d in descs:
                d.wait()

            def scale_body(g, carry2):
                vvec = v_all[pl.ds(f * BW + g * 16, 16)]
                for e in range(E):
                    stg[e, pl.ds(g * 16, 16)] = (
                        stg[e, pl.ds(g * 16, 16)] * vvec)
                return carry2

            lax.fori_loop(0, BW // 16, scale_body, 0)

            odescs = []
            for et in range(ET):
                for k in range(nbt):
                    base = (f * ET + et) * ntile + wid * nbt + k
                    odescs.append(pltpu.async_copy(
                        stg.at[pl.ds(et * 8, 8), pl.ds(k * 128, 128)],
                        out_hbm.at[base],
                        sem_o))
            for d in odescs:
                d.wait()
            return carry

        lax.fori_loop(0, F, f_body, 0)

    return emb_kernel(xf, vf, wt)


def kernel(x, v, W):
    B, F = x.shape
    NF, E = W.shape
    xf = x.T.reshape(B * F)
    vf = v.T.reshape(B * F)
    wt = W.T
    out_lin = _run(B, F, E, NF, xf, vf, wt)
    out5 = out_lin.reshape(F, E // 8, B // 128, 8, 128)
    return out5.transpose(2, 4, 0, 1, 3).reshape(B, F, E)
